# baseline (speedup 1.0000x reference)
import functools

import jax
import jax.numpy as jnp
from jax import lax
from jax.experimental import pallas as pl
from jax.experimental.pallas import tpu as pltpu, tpu_sc as plsc

_LANES = 16
_WROWS = 32
_LC = 128


def _make_gather(B, C, num_workers, num_cores):
    b_per_w = B // num_workers
    n_win = b_per_w // _WROWS
    mesh = plsc.VectorSubcoreMesh(core_axis_name="c", subcore_axis_name="s")

    @functools.partial(
        pl.kernel,
        out_type=jax.ShapeDtypeStruct((B,), jnp.float32),
        mesh=mesh,
        scratch_types=[
            pltpu.VMEM((b_per_w,), jnp.int32),
            pltpu.VMEM((_WROWS, C), jnp.float32),
            pltpu.VMEM((_WROWS, C), jnp.float32),
            pltpu.VMEM((b_per_w,), jnp.float32),
            pltpu.SemaphoreType.DMA,
            pltpu.SemaphoreType.DMA,
        ],
        compiler_params=pltpu.CompilerParams(
            use_tc_tiling_on_sc=True, needs_layout_passes=False
        ),
    )
    def gather_kernel(x_hbm, y_hbm, out_hbm, y_v, buf0, buf1, vals_v, s0, s1):
        wid = lax.axis_index("s") * num_cores + lax.axis_index("c")
        base = wid * b_per_w
        pltpu.sync_copy(y_hbm.at[pl.ds(base, b_per_w)], y_v)
        bufs = (buf0, buf1)
        sems = (s0, s1)

        def fetch(w):
            r0 = base + w * _WROWS
            descs = []
            for g in range(0, C, _LC):
                cw = min(_LC, C - g)
                descs.append(
                    pltpu.async_copy(
                        x_hbm.at[pl.ds(r0, _WROWS), pl.ds(g, cw)],
                        bufs[w % 2].at[:, pl.ds(g, cw)],
                        sems[w % 2],
                    )
                )
            return descs

        def select(w):
            buf = bufs[w % 2]
            for i in range(_WROWS // _LANES):
                off = w * _WROWS + i * _LANES
                rows_v = lax.iota(jnp.int32, _LANES) + i * _LANES
                cols = y_v[pl.ds(off, _LANES)]
                vals_v[pl.ds(off, _LANES)] = plsc.load_gather(
                    buf, [rows_v, cols]
                )

        prev = fetch(0)
        for w in range(1, n_win + 1):
            cur = fetch(w) if w < n_win else None
            for d in prev:
                d.wait()
            select(w - 1)
            prev = cur
        pltpu.sync_copy(vals_v, out_hbm.at[pl.ds(base, b_per_w)])

    return gather_kernel


def kernel(logits, y):
    B, C = logits.shape
    info = plsc.get_sparse_core_info()
    num_workers = info.num_cores * info.num_subcores
    y32 = y.astype(jnp.int32)
    return _make_gather(B, C, num_workers, info.num_cores)(logits, y32)


# speedup vs baseline: 1.0301x; 1.0301x over previous
import functools

import jax
import jax.numpy as jnp
from jax import lax
from jax.experimental import pallas as pl
from jax.experimental.pallas import tpu as pltpu, tpu_sc as plsc

_LANES = 16
_WROWS = 32
_NBUF = 3


def _make_gather(B, C, num_workers, num_cores):
    b_per_w = B // num_workers
    n_win = b_per_w // _WROWS
    mesh = plsc.VectorSubcoreMesh(core_axis_name="c", subcore_axis_name="s")

    @functools.partial(
        pl.kernel,
        out_type=jax.ShapeDtypeStruct((B,), jnp.float32),
        mesh=mesh,
        scratch_types=[
            pltpu.VMEM((b_per_w,), jnp.int32),
        ] + [pltpu.VMEM((_WROWS, C), jnp.float32) for _ in range(_NBUF)] + [
            pltpu.VMEM((b_per_w,), jnp.float32),
        ] + [pltpu.SemaphoreType.DMA for _ in range(_NBUF)],
        compiler_params=pltpu.CompilerParams(
            use_tc_tiling_on_sc=True, needs_layout_passes=False
        ),
    )
    def gather_kernel(x_hbm, y_hbm, out_hbm, y_v, *rest):
        bufs = rest[:_NBUF]
        vals_v = rest[_NBUF]
        sems = rest[_NBUF + 1:]
        wid = lax.axis_index("s") * num_cores + lax.axis_index("c")
        base = wid * b_per_w
        pltpu.sync_copy(y_hbm.at[pl.ds(base, b_per_w)], y_v)

        def fetch(w):
            return pltpu.async_copy(
                x_hbm.at[pl.ds(base + w * _WROWS, _WROWS), :],
                bufs[w % _NBUF],
                sems[w % _NBUF],
            )

        def select(w):
            buf = bufs[w % _NBUF]
            for i in range(_WROWS // _LANES):
                off = w * _WROWS + i * _LANES
                rows_v = lax.iota(jnp.int32, _LANES) + i * _LANES
                cols = y_v[pl.ds(off, _LANES)]
                vals_v[pl.ds(off, _LANES)] = plsc.load_gather(
                    buf, [rows_v, cols]
                )

        descs = [fetch(w) for w in range(min(_NBUF, n_win))]
        for w in range(n_win):
            descs[w % _NBUF].wait()
            select(w)
            if w + _NBUF < n_win:
                descs[w % _NBUF] = fetch(w + _NBUF)
        pltpu.sync_copy(vals_v, out_hbm.at[pl.ds(base, b_per_w)])

    return gather_kernel


def kernel(logits, y):
    B, C = logits.shape
    info = plsc.get_sparse_core_info()
    num_workers = info.num_cores * info.num_subcores
    y32 = y.astype(jnp.int32)
    return _make_gather(B, C, num_workers, info.num_cores)(logits, y32)


# speedup vs baseline: 1.0639x; 1.0328x over previous
import functools

import jax
import jax.numpy as jnp
from jax import lax
from jax.experimental import pallas as pl
from jax.experimental.pallas import tpu as pltpu, tpu_sc as plsc

_LANES = 16
_WROWS = 32
_BR = 2048
_LC = 128
_SPLIT = 10240


def _make_sc_gather(B, C, row0, num_workers, num_cores):
    rows = B - row0
    b_per_w = rows // num_workers
    n_win = b_per_w // _WROWS
    mesh = plsc.VectorSubcoreMesh(core_axis_name="c", subcore_axis_name="s")

    @functools.partial(
        pl.kernel,
        out_type=jax.ShapeDtypeStruct((rows,), jnp.float32),
        mesh=mesh,
        scratch_types=[
            pltpu.VMEM((b_per_w,), jnp.int32),
            pltpu.VMEM((_WROWS, C), jnp.float32),
            pltpu.VMEM((_WROWS, C), jnp.float32),
            pltpu.VMEM((b_per_w,), jnp.float32),
            pltpu.SemaphoreType.DMA,
            pltpu.SemaphoreType.DMA,
        ],
        compiler_params=pltpu.CompilerParams(
            use_tc_tiling_on_sc=True, needs_layout_passes=False
        ),
    )
    def gather_kernel(x_hbm, y_hbm, out_hbm, y_v, buf0, buf1, vals_v, s0, s1):
        wid = lax.axis_index("s") * num_cores + lax.axis_index("c")
        base = wid * b_per_w
        pltpu.sync_copy(y_hbm.at[pl.ds(row0 + base, b_per_w)], y_v)
        bufs = (buf0, buf1)
        sems = (s0, s1)

        def select(w):
            buf = bufs[w % 2]
            for i in range(_WROWS // _LANES):
                off = w * _WROWS + i * _LANES
                rows_v = lax.iota(jnp.int32, _LANES) + i * _LANES
                cols = y_v[pl.ds(off, _LANES)]
                vals_v[pl.ds(off, _LANES)] = plsc.load_gather(
                    buf, [rows_v, cols]
                )

        descs = [None, None]
        for w in range(n_win):
            descs[w % 2] = pltpu.async_copy(
                x_hbm.at[pl.ds(row0 + base + w * _WROWS, _WROWS), :],
                bufs[w % 2],
                sems[w % 2],
            )
            if w >= 1:
                descs[(w - 1) % 2].wait()
                select(w - 1)
        descs[(n_win - 1) % 2].wait()
        select(n_win - 1)
        pltpu.sync_copy(vals_v, out_hbm.at[pl.ds(base, b_per_w)])

    return gather_kernel


def _tc_select_kernel(y_ref, x_ref, o_ref):
    BR, C = x_ref.shape
    yb = y_ref[...].reshape(BR, 1)
    acc = jnp.zeros((BR, _LC), jnp.float32)
    for k in range(0, C, _LC):
        w = min(_LC, C - k)
        ids = jax.lax.broadcasted_iota(jnp.int32, (BR, w), 1) + k
        hit = jnp.where(ids == yb, x_ref[:, k:k + w], 0.0)
        if w < _LC:
            hit = jnp.pad(hit, ((0, 0), (0, _LC - w)))
        acc = acc + hit
    o_ref[...] = jnp.sum(acc, axis=1)


def _tc_select(logits, y32, n_rows):
    C = logits.shape[1]
    return pl.pallas_call(
        _tc_select_kernel,
        grid=(n_rows // _BR,),
        in_specs=[
            pl.BlockSpec((_BR,), lambda i: (i,)),
            pl.BlockSpec((_BR, C), lambda i: (i, 0)),
        ],
        out_specs=pl.BlockSpec((_BR,), lambda i: (i,)),
        out_shape=jax.ShapeDtypeStruct((n_rows,), jnp.float32),
    )(y32, logits)


def kernel(logits, y):
    B, C = logits.shape
    y32 = y.astype(jnp.int32)
    info = plsc.get_sparse_core_info()
    num_workers = info.num_cores * info.num_subcores
    sc_part = _make_sc_gather(B, C, _SPLIT, num_workers, info.num_cores)(
        logits, y32
    )
    tc_part = _tc_select(logits, y32, _SPLIT)
    return jnp.concatenate([tc_part, sc_part])
